# baseline (speedup 1.0000x reference)
import functools

import jax
import jax.numpy as jnp
from jax import lax
from jax.experimental import pallas as pl
from jax.experimental.pallas import tpu as pltpu
from jax.experimental.pallas import tpu_sc as plsc

_K = 8192
_D = 32
_N = 6272

_M_TILE = 784
_K_TILE = 1024
_MT = _N // _M_TILE
_KT = _K // _K_TILE


def _conv(x, w, b, stride):
    y = jax.lax.conv_general_dilated(
        x, w, window_strides=(stride, stride), padding=((1, 1), (1, 1)),
        dimension_numbers=("NCHW", "OIHW", "NCHW"))
    return y + b.reshape(1, -1, 1, 1)


def _conv_transpose(x, w, b):
    wf = jnp.flip(w, axis=(2, 3)).transpose(1, 0, 2, 3)
    y = jax.lax.conv_general_dilated(
        x, wf, window_strides=(1, 1), padding=((2, 2), (2, 2)),
        lhs_dilation=(2, 2), dimension_numbers=("NCHW", "OIHW", "NCHW"))
    return y + b.reshape(1, -1, 1, 1)


_RC = 8
_NCHUNK = _M_TILE // _RC
_LANES = 128
_NG = _K_TILE // _LANES


def _vq_argmin_body(flat_ref, fsq_ref, cb_ref, csq_ref, q_ref, loss_ref,
                    mm_ref, aval_ref, aidx_ref, acc_ref):
    m = pl.program_id(0)
    k = pl.program_id(1)
    mm_ref[...] = lax.dot_general(flat_ref[...], cb_ref[...],
                                  (((1,), (1,)), ((), ())),
                                  preferred_element_type=jnp.float32)
    csq = csq_ref[...]

    @pl.when(k == 0)
    def _init():
        aval_ref[...] = jnp.full((_M_TILE, _LANES), jnp.inf, jnp.float32)
        aidx_ref[...] = jnp.zeros((_M_TILE, _LANES), jnp.int32)

    def chunk_body(c, carry):
        r0 = c * _RC
        base = fsq_ref[pl.ds(r0, _RC), :] + csq
        dist = base - 2.0 * mm_ref[pl.ds(r0, _RC), :]
        av = aval_ref[pl.ds(r0, _RC), :]
        ai = aidx_ref[pl.ds(r0, _RC), :]
        for g in range(_NG):
            dg = lax.slice(dist, (0, g * _LANES), (_RC, (g + 1) * _LANES))
            upd = dg < av
            av = jnp.where(upd, dg, av)
            ai = jnp.where(upd, k * _K_TILE + g * _LANES, ai)
        aval_ref[pl.ds(r0, _RC), :] = av
        aidx_ref[pl.ds(r0, _RC), :] = ai
        return carry

    lax.fori_loop(0, _NCHUNK, chunk_body, 0, unroll=2)

    @pl.when(k == _KT - 1)
    def _finish():
        def fin_body(c, s):
            r0 = c * _RC
            av = aval_ref[pl.ds(r0, _RC), :]
            ai = aidx_ref[pl.ds(r0, _RC), :]
            gmin = jnp.min(av, axis=1, keepdims=True)
            lane = lax.broadcasted_iota(jnp.int32, (_RC, _LANES), 1)
            cand = jnp.where(av == gmin, ai + lane, jnp.int32(2 ** 30))
            q_ref[pl.ds(r0, _RC), :] = jnp.min(cand, axis=1, keepdims=True)
            return s + jnp.sum(gmin)

        s = lax.fori_loop(0, _NCHUNK, fin_body, jnp.float32(0.0))
        prev = jnp.where(m == 0, 0.0, acc_ref[0, 0])
        acc_ref[0, 0] = prev + s

        @pl.when(m == _MT - 1)
        def _emit():
            loss_ref[0, 0] = acc_ref[0, 0]


def _vq_argmin(flat, flat_sq, codebook, cb_sq):
    q, loss_sum = pl.pallas_call(
        _vq_argmin_body,
        grid=(_MT, _KT),
        in_specs=[
            pl.BlockSpec((_M_TILE, _D), lambda m, k: (m, 0)),
            pl.BlockSpec((_M_TILE, 1), lambda m, k: (m, 0)),
            pl.BlockSpec((_K_TILE, _D), lambda m, k: (k, 0)),
            pl.BlockSpec((1, _K_TILE), lambda m, k: (0, k)),
        ],
        out_specs=[
            pl.BlockSpec((_M_TILE, 1), lambda m, k: (m, 0)),
            pl.BlockSpec(memory_space=pltpu.SMEM),
        ],
        out_shape=[
            jax.ShapeDtypeStruct((_N, 1), jnp.int32),
            jax.ShapeDtypeStruct((1, 1), jnp.float32),
        ],
        scratch_shapes=[
            pltpu.VMEM((_M_TILE, _K_TILE), jnp.float32),
            pltpu.VMEM((_M_TILE, _LANES), jnp.float32),
            pltpu.VMEM((_M_TILE, _LANES), jnp.int32),
            pltpu.SMEM((1, 1), jnp.float32),
        ],
    )(flat, flat_sq, codebook, cb_sq)
    return q.reshape(_N), loss_sum[0, 0]


_CHUNK = 112
_CHUNKS_PER_W = 2
_B_PER_W = _CHUNK * _CHUNKS_PER_W
_NW = 32
_N_PAD = _B_PER_W * _NW


def _sc_gather_body(table_hbm, idx_hbm, out_hbm, idx_v, rows_v, sem):
    wid = lax.axis_index("s") * 2 + lax.axis_index("c")
    pltpu.sync_copy(idx_hbm.at[pl.ds(wid * _CHUNKS_PER_W, _CHUNKS_PER_W)],
                    idx_v)
    for j in range(_CHUNKS_PER_W):
        pltpu.async_copy(table_hbm.at[idx_v.at[j]], rows_v.at[j], sem).wait()
    pltpu.sync_copy(rows_v,
                    out_hbm.at[pl.ds(wid * _CHUNKS_PER_W, _CHUNKS_PER_W)])


def _sc_gather(codebook, q):
    q_pad = jnp.concatenate(
        [q, jnp.zeros((_N_PAD - _N,), dtype=jnp.int32)]).reshape(
            _NW * _CHUNKS_PER_W, _CHUNK)
    mesh = plsc.VectorSubcoreMesh(core_axis_name="c", subcore_axis_name="s")
    gathered = pl.kernel(
        _sc_gather_body,
        mesh=mesh,
        out_type=jax.ShapeDtypeStruct((_NW * _CHUNKS_PER_W, _CHUNK, _D),
                                      jnp.float32),
        scratch_types=[
            pltpu.VMEM((_CHUNKS_PER_W, _CHUNK), jnp.int32),
            pltpu.VMEM((_CHUNKS_PER_W, _CHUNK, _D), jnp.float32),
            pltpu.SemaphoreType.DMA,
        ],
        compiler_params=pltpu.CompilerParams(use_tc_tiling_on_sc=False),
    )(codebook, q_pad)
    return gathered.reshape(_N_PAD, _D)[:_N]


def kernel(imgs, enc_w1, enc_b1, enc_w2, enc_b2, codebook, dec_w1, dec_b1,
           dec_w2, dec_b2):
    z_e = jax.nn.relu(_conv(imgs, enc_w1, enc_b1, 2))
    z_e = jax.nn.relu(_conv(z_e, enc_w2, enc_b2, 2))
    z = jnp.transpose(z_e, (0, 2, 3, 1))
    z_shape = z.shape
    flat = z.reshape(-1, _D)

    flat_sq = jnp.sum(flat ** 2, axis=1, keepdims=True)
    cb_sq = jnp.sum(codebook ** 2, axis=1).reshape(1, _K)

    q, loss_sum = _vq_argmin(flat, flat_sq, codebook, cb_sq)
    z_q_flat = _sc_gather(codebook, q)

    vq_loss = loss_sum * (2.0 / (_N * _D))

    z_q = z + (z_q_flat.reshape(z_shape) - z)
    encoded = jnp.transpose(z_q, (0, 3, 1, 2))
    d = jax.nn.relu(_conv_transpose(encoded, dec_w1, dec_b1))
    decoded = jax.nn.relu(_conv_transpose(d, dec_w2, dec_b2))
    return encoded, decoded, vq_loss


# speedup vs baseline: 1.8958x; 1.8958x over previous
import functools

import jax
import jax.numpy as jnp
from jax import lax
from jax.experimental import pallas as pl
from jax.experimental.pallas import tpu as pltpu
from jax.experimental.pallas import tpu_sc as plsc

_K = 8192
_D = 32
_N = 6272

_M_TILE = 784
_K_TILE = 1024
_MT = _N // _M_TILE
_KT = _K // _K_TILE


def _conv(x, w, b, stride):
    y = jax.lax.conv_general_dilated(
        x, w, window_strides=(stride, stride), padding=((1, 1), (1, 1)),
        dimension_numbers=("NCHW", "OIHW", "NCHW"))
    return y + b.reshape(1, -1, 1, 1)


def _conv_transpose(x, w, b):
    wf = jnp.flip(w, axis=(2, 3)).transpose(1, 0, 2, 3)
    y = jax.lax.conv_general_dilated(
        x, wf, window_strides=(1, 1), padding=((2, 2), (2, 2)),
        lhs_dilation=(2, 2), dimension_numbers=("NCHW", "OIHW", "NCHW"))
    return y + b.reshape(1, -1, 1, 1)


_RC = 8
_NCHUNK = _M_TILE // _RC
_LANES = 128
_NG = _K_TILE // _LANES


def _vq_argmin_body(flat_ref, fsq_ref, cb_ref, csq_ref, q_ref, loss_ref,
                    mm_ref, aval_ref, aidx_ref, acc_ref):
    m = pl.program_id(0)
    k = pl.program_id(1)
    mm_ref[...] = lax.dot_general(flat_ref[...], cb_ref[...],
                                  (((1,), (1,)), ((), ())),
                                  preferred_element_type=jnp.float32)
    csq = csq_ref[...]

    @pl.when(k == 0)
    def _init():
        aval_ref[...] = jnp.full((_M_TILE, _LANES), jnp.inf, jnp.float32)
        aidx_ref[...] = jnp.zeros((_M_TILE, _LANES), jnp.int32)

    for c in range(_NCHUNK):
        r0 = c * _RC
        base = fsq_ref[r0:r0 + _RC, :] + csq
        dist = base - 2.0 * mm_ref[r0:r0 + _RC, :]
        pairs = []
        for g in range(_NG):
            dg = lax.slice(dist, (0, g * _LANES), (_RC, (g + 1) * _LANES))
            pairs.append((dg, jnp.full((_RC, _LANES), g * _LANES,
                                       jnp.int32)))
        while len(pairs) > 1:
            nxt = []
            for a in range(0, len(pairs), 2):
                (va, ia), (vb, ib) = pairs[a], pairs[a + 1]
                take_b = vb < va
                nxt.append((jnp.where(take_b, vb, va),
                            jnp.where(take_b, ib, ia)))
            pairs = nxt
        lv, li = pairs[0]
        av = aval_ref[r0:r0 + _RC, :]
        ai = aidx_ref[r0:r0 + _RC, :]
        upd = lv < av
        aval_ref[r0:r0 + _RC, :] = jnp.where(upd, lv, av)
        aidx_ref[r0:r0 + _RC, :] = jnp.where(upd, li + k * _K_TILE, ai)

    @pl.when(k == _KT - 1)
    def _finish():
        s = jnp.float32(0.0)
        for c in range(_NCHUNK):
            r0 = c * _RC
            av = aval_ref[r0:r0 + _RC, :]
            ai = aidx_ref[r0:r0 + _RC, :]
            gmin = jnp.min(av, axis=1, keepdims=True)
            lane = lax.broadcasted_iota(jnp.int32, (_RC, _LANES), 1)
            cand = jnp.where(av == gmin, ai + lane, jnp.int32(2 ** 30))
            q_ref[r0:r0 + _RC, :] = jnp.min(cand, axis=1, keepdims=True)
            s = s + jnp.sum(gmin)

        prev = jnp.where(m == 0, 0.0, acc_ref[0, 0])
        acc_ref[0, 0] = prev + s

        @pl.when(m == _MT - 1)
        def _emit():
            loss_ref[0, 0] = acc_ref[0, 0]


def _vq_argmin(flat, flat_sq, codebook, cb_sq):
    q, loss_sum = pl.pallas_call(
        _vq_argmin_body,
        grid=(_MT, _KT),
        in_specs=[
            pl.BlockSpec((_M_TILE, _D), lambda m, k: (m, 0)),
            pl.BlockSpec((_M_TILE, 1), lambda m, k: (m, 0)),
            pl.BlockSpec((_K_TILE, _D), lambda m, k: (k, 0)),
            pl.BlockSpec((1, _K_TILE), lambda m, k: (0, k)),
        ],
        out_specs=[
            pl.BlockSpec((_M_TILE, 1), lambda m, k: (m, 0)),
            pl.BlockSpec(memory_space=pltpu.SMEM),
        ],
        out_shape=[
            jax.ShapeDtypeStruct((_N, 1), jnp.int32),
            jax.ShapeDtypeStruct((1, 1), jnp.float32),
        ],
        scratch_shapes=[
            pltpu.VMEM((_M_TILE, _K_TILE), jnp.float32),
            pltpu.VMEM((_M_TILE, _LANES), jnp.float32),
            pltpu.VMEM((_M_TILE, _LANES), jnp.int32),
            pltpu.SMEM((1, 1), jnp.float32),
        ],
    )(flat, flat_sq, codebook, cb_sq)
    return q.reshape(_N), loss_sum[0, 0]


_CHUNK = 112
_CHUNKS_PER_W = 2
_B_PER_W = _CHUNK * _CHUNKS_PER_W
_NW = 32
_N_PAD = _B_PER_W * _NW


def _sc_gather_body(table_hbm, idx_hbm, out_hbm, idx_v, rows_v, sem):
    wid = lax.axis_index("s") * 2 + lax.axis_index("c")
    pltpu.sync_copy(idx_hbm.at[pl.ds(wid * _CHUNKS_PER_W, _CHUNKS_PER_W)],
                    idx_v)
    for j in range(_CHUNKS_PER_W):
        pltpu.async_copy(table_hbm.at[idx_v.at[j]], rows_v.at[j], sem).wait()
    pltpu.sync_copy(rows_v,
                    out_hbm.at[pl.ds(wid * _CHUNKS_PER_W, _CHUNKS_PER_W)])


def _sc_gather(codebook, q):
    q_pad = jnp.concatenate(
        [q, jnp.zeros((_N_PAD - _N,), dtype=jnp.int32)]).reshape(
            _NW * _CHUNKS_PER_W, _CHUNK)
    mesh = plsc.VectorSubcoreMesh(core_axis_name="c", subcore_axis_name="s")
    gathered = pl.kernel(
        _sc_gather_body,
        mesh=mesh,
        out_type=jax.ShapeDtypeStruct((_NW * _CHUNKS_PER_W, _CHUNK, _D),
                                      jnp.float32),
        scratch_types=[
            pltpu.VMEM((_CHUNKS_PER_W, _CHUNK), jnp.int32),
            pltpu.VMEM((_CHUNKS_PER_W, _CHUNK, _D), jnp.float32),
            pltpu.SemaphoreType.DMA,
        ],
        compiler_params=pltpu.CompilerParams(use_tc_tiling_on_sc=False),
    )(codebook, q_pad)
    return gathered.reshape(_N_PAD, _D)[:_N]


def kernel(imgs, enc_w1, enc_b1, enc_w2, enc_b2, codebook, dec_w1, dec_b1,
           dec_w2, dec_b2):
    z_e = jax.nn.relu(_conv(imgs, enc_w1, enc_b1, 2))
    z_e = jax.nn.relu(_conv(z_e, enc_w2, enc_b2, 2))
    z = jnp.transpose(z_e, (0, 2, 3, 1))
    z_shape = z.shape
    flat = z.reshape(-1, _D)

    flat_sq = jnp.sum(flat ** 2, axis=1, keepdims=True)
    cb_sq = jnp.sum(codebook ** 2, axis=1).reshape(1, _K)

    q, loss_sum = _vq_argmin(flat, flat_sq, codebook, cb_sq)
    z_q_flat = _sc_gather(codebook, q)

    vq_loss = loss_sum * (2.0 / (_N * _D))

    z_q = z + (z_q_flat.reshape(z_shape) - z)
    encoded = jnp.transpose(z_q, (0, 3, 1, 2))
    d = jax.nn.relu(_conv_transpose(encoded, dec_w1, dec_b1))
    decoded = jax.nn.relu(_conv_transpose(d, dec_w2, dec_b2))
    return encoded, decoded, vq_loss


# speedup vs baseline: 2.1192x; 1.1178x over previous
import functools

import jax
import jax.numpy as jnp
from jax import lax
from jax.experimental import pallas as pl
from jax.experimental.pallas import tpu as pltpu
from jax.experimental.pallas import tpu_sc as plsc

_K = 8192
_D = 32
_N = 6272

_M_TILE = 784
_K_TILE = 1024
_MT = _N // _M_TILE
_KT = _K // _K_TILE


def _conv(x, w, b, stride):
    y = jax.lax.conv_general_dilated(
        x, w, window_strides=(stride, stride), padding=((1, 1), (1, 1)),
        dimension_numbers=("NCHW", "OIHW", "NCHW"))
    return y + b.reshape(1, -1, 1, 1)


def _conv_transpose(x, w, b):
    wf = jnp.flip(w, axis=(2, 3)).transpose(1, 0, 2, 3)
    y = jax.lax.conv_general_dilated(
        x, wf, window_strides=(1, 1), padding=((2, 2), (2, 2)),
        lhs_dilation=(2, 2), dimension_numbers=("NCHW", "OIHW", "NCHW"))
    return y + b.reshape(1, -1, 1, 1)


_RC = 8
_NCHUNK = _M_TILE // _RC
_LANES = 128
_NG = _K_TILE // _LANES


def _vq_argmin_body(flat2_ref, fsq_ref, cb_ref, csq_ref, q_ref, loss_ref,
                    mm_ref, aval_ref, aidx_ref, fsqb_ref, acc_ref):
    m = pl.program_id(0)
    k = pl.program_id(1)
    mm_ref[...] = lax.dot_general(flat2_ref[...], cb_ref[...],
                                  (((1,), (1,)), ((), ())),
                                  preferred_element_type=jnp.float32)

    @pl.when(k == 0)
    def _init():
        aval_ref[...] = jnp.full((_M_TILE, _LANES), jnp.inf, jnp.float32)
        fsqb_ref[...] = jnp.broadcast_to(fsq_ref[...], (_M_TILE, _LANES))

    csq8 = jnp.broadcast_to(csq_ref[...], (_RC, _K_TILE))
    csq_sl = [lax.slice(csq8, (0, g * _LANES), (_RC, (g + 1) * _LANES))
              for g in range(_NG)]

    for c in range(_NCHUNK):
        r0 = c * _RC
        fb = fsqb_ref[r0:r0 + _RC, :]
        mm2 = mm_ref[r0:r0 + _RC, :]
        pairs = []
        for g in range(_NG):
            mg = lax.slice(mm2, (0, g * _LANES), (_RC, (g + 1) * _LANES))
            dg = (fb + csq_sl[g]) - mg
            pairs.append((dg, jnp.int32(g * _LANES)))
        lvl = []
        for a in range(0, _NG, 2):
            (va, ia), (vb, ib) = pairs[a], pairs[a + 1]
            take_b = vb < va
            lvl.append((jnp.where(take_b, vb, va),
                        jnp.where(take_b, ib, ia)))
        while len(lvl) > 1:
            nxt = []
            for a in range(0, len(lvl), 2):
                (va, ia), (vb, ib) = lvl[a], lvl[a + 1]
                take_b = vb < va
                nxt.append((jnp.where(take_b, vb, va),
                            jnp.where(take_b, ib, ia)))
            lvl = nxt
        lv, li = lvl[0]
        av = aval_ref[r0:r0 + _RC, :]
        ai = aidx_ref[r0:r0 + _RC, :]
        upd = lv < av
        aval_ref[r0:r0 + _RC, :] = jnp.where(upd, lv, av)
        aidx_ref[r0:r0 + _RC, :] = jnp.where(upd, li + k * _K_TILE, ai)

    @pl.when(k == _KT - 1)
    def _finish():
        av = aval_ref[...]
        ai = aidx_ref[...]
        gmin = jnp.min(av, axis=1, keepdims=True)
        lane = lax.broadcasted_iota(jnp.int32, (_M_TILE, _LANES), 1)
        cand = jnp.where(av == gmin, ai + lane, jnp.int32(2 ** 30))
        q_ref[...] = jnp.min(cand, axis=1, keepdims=True)
        s = jnp.sum(gmin)

        prev = jnp.where(m == 0, 0.0, acc_ref[0, 0])
        acc_ref[0, 0] = prev + s

        @pl.when(m == _MT - 1)
        def _emit():
            loss_ref[0, 0] = acc_ref[0, 0]


def _vq_argmin(flat2, flat_sq, codebook, cb_sq):
    q, loss_sum = pl.pallas_call(
        _vq_argmin_body,
        grid=(_MT, _KT),
        in_specs=[
            pl.BlockSpec((_M_TILE, _D), lambda m, k: (m, 0)),
            pl.BlockSpec((_M_TILE, 1), lambda m, k: (m, 0)),
            pl.BlockSpec((_K_TILE, _D), lambda m, k: (k, 0)),
            pl.BlockSpec((1, _K_TILE), lambda m, k: (0, k)),
        ],
        out_specs=[
            pl.BlockSpec((_M_TILE, 1), lambda m, k: (m, 0)),
            pl.BlockSpec(memory_space=pltpu.SMEM),
        ],
        out_shape=[
            jax.ShapeDtypeStruct((_N, 1), jnp.int32),
            jax.ShapeDtypeStruct((1, 1), jnp.float32),
        ],
        scratch_shapes=[
            pltpu.VMEM((_M_TILE, _K_TILE), jnp.float32),
            pltpu.VMEM((_M_TILE, _LANES), jnp.float32),
            pltpu.VMEM((_M_TILE, _LANES), jnp.int32),
            pltpu.VMEM((_M_TILE, _LANES), jnp.float32),
            pltpu.SMEM((1, 1), jnp.float32),
        ],
    )(flat2, flat_sq, codebook, cb_sq)
    return q.reshape(_N), loss_sum[0, 0]


_CHUNK = 112
_CHUNKS_PER_W = 2
_B_PER_W = _CHUNK * _CHUNKS_PER_W
_NW = 32
_N_PAD = _B_PER_W * _NW


def _sc_gather_body(table_hbm, idx_hbm, out_hbm, idx_v, rows_v, sem):
    wid = lax.axis_index("s") * 2 + lax.axis_index("c")
    pltpu.sync_copy(idx_hbm.at[pl.ds(wid * _CHUNKS_PER_W, _CHUNKS_PER_W)],
                    idx_v)
    for j in range(_CHUNKS_PER_W):
        pltpu.async_copy(table_hbm.at[idx_v.at[j]], rows_v.at[j], sem).wait()
    pltpu.sync_copy(rows_v,
                    out_hbm.at[pl.ds(wid * _CHUNKS_PER_W, _CHUNKS_PER_W)])


def _sc_gather(codebook, q):
    q_pad = jnp.concatenate(
        [q, jnp.zeros((_N_PAD - _N,), dtype=jnp.int32)]).reshape(
            _NW * _CHUNKS_PER_W, _CHUNK)
    mesh = plsc.VectorSubcoreMesh(core_axis_name="c", subcore_axis_name="s")
    gathered = pl.kernel(
        _sc_gather_body,
        mesh=mesh,
        out_type=jax.ShapeDtypeStruct((_NW * _CHUNKS_PER_W, _CHUNK, _D),
                                      jnp.float32),
        scratch_types=[
            pltpu.VMEM((_CHUNKS_PER_W, _CHUNK), jnp.int32),
            pltpu.VMEM((_CHUNKS_PER_W, _CHUNK, _D), jnp.float32),
            pltpu.SemaphoreType.DMA,
        ],
        compiler_params=pltpu.CompilerParams(use_tc_tiling_on_sc=False),
    )(codebook, q_pad)
    return gathered.reshape(_N_PAD, _D)[:_N]


def kernel(imgs, enc_w1, enc_b1, enc_w2, enc_b2, codebook, dec_w1, dec_b1,
           dec_w2, dec_b2):
    z_e = jax.nn.relu(_conv(imgs, enc_w1, enc_b1, 2))
    z_e = jax.nn.relu(_conv(z_e, enc_w2, enc_b2, 2))
    z = jnp.transpose(z_e, (0, 2, 3, 1))
    z_shape = z.shape
    flat = z.reshape(-1, _D)

    flat_sq = jnp.sum(flat ** 2, axis=1, keepdims=True)
    cb_sq = jnp.sum(codebook ** 2, axis=1).reshape(1, _K)

    q, loss_sum = _vq_argmin(flat * 2.0, flat_sq, codebook, cb_sq)
    z_q_flat = _sc_gather(codebook, q)

    vq_loss = loss_sum * (2.0 / (_N * _D))

    z_q = z + (z_q_flat.reshape(z_shape) - z)
    encoded = jnp.transpose(z_q, (0, 3, 1, 2))
    d = jax.nn.relu(_conv_transpose(encoded, dec_w1, dec_b1))
    decoded = jax.nn.relu(_conv_transpose(d, dec_w2, dec_b2))
    return encoded, decoded, vq_loss


# speedup vs baseline: 2.3655x; 1.1162x over previous
import functools

import jax
import jax.numpy as jnp
from jax import lax
from jax.experimental import pallas as pl
from jax.experimental.pallas import tpu as pltpu
from jax.experimental.pallas import tpu_sc as plsc

_K = 8192
_D = 32
_N = 6272

_M_TILE = 784
_K_TILE = 1024
_MT = _N // _M_TILE
_KT = _K // _K_TILE


def _conv(x, w, b, stride):
    y = jax.lax.conv_general_dilated(
        x, w, window_strides=(stride, stride), padding=((1, 1), (1, 1)),
        dimension_numbers=("NCHW", "OIHW", "NCHW"))
    return y + b.reshape(1, -1, 1, 1)


def _conv_transpose(x, w, b):
    wf = jnp.flip(w, axis=(2, 3)).transpose(1, 0, 2, 3)
    y = jax.lax.conv_general_dilated(
        x, wf, window_strides=(1, 1), padding=((2, 2), (2, 2)),
        lhs_dilation=(2, 2), dimension_numbers=("NCHW", "OIHW", "NCHW"))
    return y + b.reshape(1, -1, 1, 1)


_RC = 8
_NCHUNK = _M_TILE // _RC
_LANES = 128
_NG = _K_TILE // _LANES


def _vq_argmin_body(flat2_ref, fsq_ref, cb_ref, csq_ref, q_ref, loss_ref,
                    mma_ref, mmb_ref, aval_ref, aidx_ref, fsqb_ref, acc_ref):
    m = pl.program_id(0)
    mm_bufs = (mma_ref, mmb_ref)

    fsqb_ref[...] = jnp.broadcast_to(fsq_ref[...], (_M_TILE, _LANES))

    def sub_matmul(s):
        cbs = cb_ref[s * _K_TILE:(s + 1) * _K_TILE, :]
        mm_bufs[s % 2][...] = lax.dot_general(
            flat2_ref[...], cbs, (((1,), (1,)), ((), ())),
            preferred_element_type=jnp.float32)

    def fold(s):
        mm_ref = mm_bufs[s % 2]
        csq8 = jnp.broadcast_to(
            csq_ref[:, s * _K_TILE:(s + 1) * _K_TILE], (_RC, _K_TILE))
        csq_sl = [lax.slice(csq8, (0, g * _LANES), (_RC, (g + 1) * _LANES))
                  for g in range(_NG)]
        for c in range(_NCHUNK):
            r0 = c * _RC
            fb = fsqb_ref[r0:r0 + _RC, :]
            mm2 = mm_ref[r0:r0 + _RC, :]
            pairs = []
            for g in range(_NG):
                mg = lax.slice(mm2, (0, g * _LANES),
                               (_RC, (g + 1) * _LANES))
                dg = (fb + csq_sl[g]) - mg
                pairs.append((dg, jnp.int32(s * _K_TILE + g * _LANES)))
            lvl = []
            for a in range(0, _NG, 2):
                (va, ia), (vb, ib) = pairs[a], pairs[a + 1]
                take_b = vb < va
                lvl.append((jnp.where(take_b, vb, va),
                            jnp.where(take_b, ib, ia)))
            while len(lvl) > 1:
                nxt = []
                for a in range(0, len(lvl), 2):
                    (va, ia), (vb, ib) = lvl[a], lvl[a + 1]
                    take_b = vb < va
                    nxt.append((jnp.where(take_b, vb, va),
                                jnp.where(take_b, ib, ia)))
                lvl = nxt
            lv, li = lvl[0]
            if s == 0:
                aval_ref[r0:r0 + _RC, :] = lv
                aidx_ref[r0:r0 + _RC, :] = li
            else:
                av = aval_ref[r0:r0 + _RC, :]
                ai = aidx_ref[r0:r0 + _RC, :]
                upd = lv < av
                aval_ref[r0:r0 + _RC, :] = jnp.where(upd, lv, av)
                aidx_ref[r0:r0 + _RC, :] = jnp.where(upd, li, ai)

    sub_matmul(0)
    for s in range(1, _KT):
        sub_matmul(s)
        fold(s - 1)
    fold(_KT - 1)

    av = aval_ref[...]
    ai = aidx_ref[...]
    gmin = jnp.min(av, axis=1, keepdims=True)
    lane = lax.broadcasted_iota(jnp.int32, (_M_TILE, _LANES), 1)
    cand = jnp.where(av == gmin, ai + lane, jnp.int32(2 ** 30))
    q_ref[...] = jnp.min(cand, axis=1, keepdims=True)
    s_loss = jnp.sum(gmin)

    prev = jnp.where(m == 0, 0.0, acc_ref[0, 0])
    acc_ref[0, 0] = prev + s_loss

    @pl.when(m == _MT - 1)
    def _emit():
        loss_ref[0, 0] = acc_ref[0, 0]


def _vq_argmin(flat2, flat_sq, codebook, cb_sq):
    q, loss_sum = pl.pallas_call(
        _vq_argmin_body,
        grid=(_MT,),
        in_specs=[
            pl.BlockSpec((_M_TILE, _D), lambda m: (m, 0)),
            pl.BlockSpec((_M_TILE, 1), lambda m: (m, 0)),
            pl.BlockSpec((_K, _D), lambda m: (0, 0)),
            pl.BlockSpec((1, _K), lambda m: (0, 0)),
        ],
        out_specs=[
            pl.BlockSpec((_M_TILE, 1), lambda m: (m, 0)),
            pl.BlockSpec(memory_space=pltpu.SMEM),
        ],
        out_shape=[
            jax.ShapeDtypeStruct((_N, 1), jnp.int32),
            jax.ShapeDtypeStruct((1, 1), jnp.float32),
        ],
        scratch_shapes=[
            pltpu.VMEM((_M_TILE, _K_TILE), jnp.float32),
            pltpu.VMEM((_M_TILE, _K_TILE), jnp.float32),
            pltpu.VMEM((_M_TILE, _LANES), jnp.float32),
            pltpu.VMEM((_M_TILE, _LANES), jnp.int32),
            pltpu.VMEM((_M_TILE, _LANES), jnp.float32),
            pltpu.SMEM((1, 1), jnp.float32),
        ],
    )(flat2, flat_sq, codebook, cb_sq)
    return q.reshape(_N), loss_sum[0, 0]


_CHUNK = 112
_CHUNKS_PER_W = 2
_B_PER_W = _CHUNK * _CHUNKS_PER_W
_NW = 32
_N_PAD = _B_PER_W * _NW


def _sc_gather_body(table_hbm, idx_hbm, out_hbm, idx_v, rows_v, sem):
    wid = lax.axis_index("s") * 2 + lax.axis_index("c")
    pltpu.sync_copy(idx_hbm.at[pl.ds(wid * _CHUNKS_PER_W, _CHUNKS_PER_W)],
                    idx_v)
    for j in range(_CHUNKS_PER_W):
        pltpu.async_copy(table_hbm.at[idx_v.at[j]], rows_v.at[j], sem).wait()
    pltpu.sync_copy(rows_v,
                    out_hbm.at[pl.ds(wid * _CHUNKS_PER_W, _CHUNKS_PER_W)])


def _sc_gather(codebook, q):
    q_pad = jnp.concatenate(
        [q, jnp.zeros((_N_PAD - _N,), dtype=jnp.int32)]).reshape(
            _NW * _CHUNKS_PER_W, _CHUNK)
    mesh = plsc.VectorSubcoreMesh(core_axis_name="c", subcore_axis_name="s")
    gathered = pl.kernel(
        _sc_gather_body,
        mesh=mesh,
        out_type=jax.ShapeDtypeStruct((_NW * _CHUNKS_PER_W, _CHUNK, _D),
                                      jnp.float32),
        scratch_types=[
            pltpu.VMEM((_CHUNKS_PER_W, _CHUNK), jnp.int32),
            pltpu.VMEM((_CHUNKS_PER_W, _CHUNK, _D), jnp.float32),
            pltpu.SemaphoreType.DMA,
        ],
        compiler_params=pltpu.CompilerParams(use_tc_tiling_on_sc=False),
    )(codebook, q_pad)
    return gathered.reshape(_N_PAD, _D)[:_N]


def kernel(imgs, enc_w1, enc_b1, enc_w2, enc_b2, codebook, dec_w1, dec_b1,
           dec_w2, dec_b2):
    z_e = jax.nn.relu(_conv(imgs, enc_w1, enc_b1, 2))
    z_e = jax.nn.relu(_conv(z_e, enc_w2, enc_b2, 2))
    z = jnp.transpose(z_e, (0, 2, 3, 1))
    z_shape = z.shape
    flat = z.reshape(-1, _D)

    flat_sq = jnp.sum(flat ** 2, axis=1, keepdims=True)
    cb_sq = jnp.sum(codebook ** 2, axis=1).reshape(1, _K)

    q, loss_sum = _vq_argmin(flat * 2.0, flat_sq, codebook, cb_sq)
    z_q_flat = _sc_gather(codebook, q)

    vq_loss = loss_sum * (2.0 / (_N * _D))

    z_q = z + (z_q_flat.reshape(z_shape) - z)
    encoded = jnp.transpose(z_q, (0, 3, 1, 2))
    d = jax.nn.relu(_conv_transpose(encoded, dec_w1, dec_b1))
    decoded = jax.nn.relu(_conv_transpose(d, dec_w2, dec_b2))
    return encoded, decoded, vq_loss


# speedup vs baseline: 2.4624x; 1.0410x over previous
import functools

import jax
import jax.numpy as jnp
from jax import lax
from jax.experimental import pallas as pl
from jax.experimental.pallas import tpu as pltpu
from jax.experimental.pallas import tpu_sc as plsc

_K = 8192
_D = 32
_N = 6272

_M_TILE = 784
_K_TILE = 1024
_MT = _N // _M_TILE
_KT = _K // _K_TILE


def _conv(x, w, b, stride):
    y = jax.lax.conv_general_dilated(
        x, w, window_strides=(stride, stride), padding=((1, 1), (1, 1)),
        dimension_numbers=("NCHW", "OIHW", "NCHW"))
    return y + b.reshape(1, -1, 1, 1)


def _conv_transpose(x, w, b):
    wf = jnp.flip(w, axis=(2, 3)).transpose(1, 0, 2, 3)
    y = jax.lax.conv_general_dilated(
        x, wf, window_strides=(1, 1), padding=((2, 2), (2, 2)),
        lhs_dilation=(2, 2), dimension_numbers=("NCHW", "OIHW", "NCHW"))
    return y + b.reshape(1, -1, 1, 1)


_RC = 8
_NCHUNK = _M_TILE // _RC
_LANES = 128
_NG = _K_TILE // _LANES


def _vq_argmin_body(flat2_ref, fsq_ref, cb_ref, csq_ref, q_ref, loss_ref,
                    mma_ref, mmb_ref, aval_ref, aidx_ref, fsqb_ref, acc_ref):
    m = pl.program_id(0)
    mm_bufs = (mma_ref, mmb_ref)

    fsqb_ref[...] = jnp.broadcast_to(fsq_ref[...], (_M_TILE, _LANES))

    flat2 = flat2_ref[...] + flat2_ref[...]

    def sub_matmul(s):
        cbs = cb_ref[s * _K_TILE:(s + 1) * _K_TILE, :]
        mm_bufs[s % 2][...] = lax.dot_general(
            flat2, cbs, (((1,), (1,)), ((), ())),
            preferred_element_type=jnp.float32)

    def fold(s):
        mm_ref = mm_bufs[s % 2]
        csq8 = jnp.broadcast_to(
            csq_ref[:, s * _K_TILE:(s + 1) * _K_TILE], (_RC, _K_TILE))
        csq_sl = [lax.slice(csq8, (0, g * _LANES), (_RC, (g + 1) * _LANES))
                  for g in range(_NG)]
        for c in range(_NCHUNK):
            r0 = c * _RC
            fb = fsqb_ref[r0:r0 + _RC, :]
            mm2 = mm_ref[r0:r0 + _RC, :]
            pairs = []
            for g in range(_NG):
                mg = lax.slice(mm2, (0, g * _LANES),
                               (_RC, (g + 1) * _LANES))
                dg = (fb + csq_sl[g]) - mg
                pairs.append((dg, jnp.int32(s * _K_TILE + g * _LANES)))
            lvl = []
            for a in range(0, _NG, 2):
                (va, ia), (vb, ib) = pairs[a], pairs[a + 1]
                take_b = vb < va
                lvl.append((jnp.where(take_b, vb, va),
                            jnp.where(take_b, ib, ia)))
            while len(lvl) > 1:
                nxt = []
                for a in range(0, len(lvl), 2):
                    (va, ia), (vb, ib) = lvl[a], lvl[a + 1]
                    take_b = vb < va
                    nxt.append((jnp.where(take_b, vb, va),
                                jnp.where(take_b, ib, ia)))
                lvl = nxt
            lv, li = lvl[0]
            if s == 0:
                aval_ref[r0:r0 + _RC, :] = lv
                aidx_ref[r0:r0 + _RC, :] = li
            else:
                av = aval_ref[r0:r0 + _RC, :]
                ai = aidx_ref[r0:r0 + _RC, :]
                upd = lv < av
                aval_ref[r0:r0 + _RC, :] = jnp.where(upd, lv, av)
                aidx_ref[r0:r0 + _RC, :] = jnp.where(upd, li, ai)

    sub_matmul(0)
    for s in range(1, _KT):
        sub_matmul(s)
        fold(s - 1)
    fold(_KT - 1)

    av = aval_ref[...]
    ai = aidx_ref[...]
    gmin = jnp.min(av, axis=1, keepdims=True)
    lane = lax.broadcasted_iota(jnp.int32, (_M_TILE, _LANES), 1)
    cand = jnp.where(av == gmin, ai + lane, jnp.int32(2 ** 30))
    q_ref[...] = jnp.min(cand, axis=1, keepdims=True)
    s_loss = jnp.sum(gmin)

    prev = jnp.where(m == 0, 0.0, acc_ref[0, 0])
    acc_ref[0, 0] = prev + s_loss

    @pl.when(m == _MT - 1)
    def _emit():
        loss_ref[0, 0] = acc_ref[0, 0]


def _vq_argmin(flat2, flat_sq, codebook, cb_sq):
    q, loss_sum = pl.pallas_call(
        _vq_argmin_body,
        grid=(_MT,),
        in_specs=[
            pl.BlockSpec((_M_TILE, _D), lambda m: (m, 0)),
            pl.BlockSpec((_M_TILE, 1), lambda m: (m, 0)),
            pl.BlockSpec((_K, _D), lambda m: (0, 0)),
            pl.BlockSpec((1, _K), lambda m: (0, 0)),
        ],
        out_specs=[
            pl.BlockSpec((_M_TILE, 1), lambda m: (m, 0)),
            pl.BlockSpec(memory_space=pltpu.SMEM),
        ],
        out_shape=[
            jax.ShapeDtypeStruct((_N, 1), jnp.int32),
            jax.ShapeDtypeStruct((1, 1), jnp.float32),
        ],
        scratch_shapes=[
            pltpu.VMEM((_M_TILE, _K_TILE), jnp.float32),
            pltpu.VMEM((_M_TILE, _K_TILE), jnp.float32),
            pltpu.VMEM((_M_TILE, _LANES), jnp.float32),
            pltpu.VMEM((_M_TILE, _LANES), jnp.int32),
            pltpu.VMEM((_M_TILE, _LANES), jnp.float32),
            pltpu.SMEM((1, 1), jnp.float32),
        ],
    )(flat2, flat_sq, codebook, cb_sq)
    return q.reshape(_N), loss_sum[0, 0]


_CHUNK = 112
_CHUNKS_PER_W = 2
_B_PER_W = _CHUNK * _CHUNKS_PER_W
_NW = 32
_N_PAD = _B_PER_W * _NW


def _sc_gather_body(table_hbm, idx_hbm, out_hbm, idx_v, rows_v, sem):
    wid = lax.axis_index("s") * 2 + lax.axis_index("c")
    pltpu.sync_copy(idx_hbm.at[pl.ds(wid * _CHUNKS_PER_W, _CHUNKS_PER_W)],
                    idx_v)
    for j in range(_CHUNKS_PER_W):
        pltpu.async_copy(table_hbm.at[idx_v.at[j]], rows_v.at[j], sem).wait()
    pltpu.sync_copy(rows_v,
                    out_hbm.at[pl.ds(wid * _CHUNKS_PER_W, _CHUNKS_PER_W)])


def _sc_gather(codebook, q):
    q_pad = jnp.concatenate(
        [q, jnp.zeros((_N_PAD - _N,), dtype=jnp.int32)]).reshape(
            _NW * _CHUNKS_PER_W, _CHUNK)
    mesh = plsc.VectorSubcoreMesh(core_axis_name="c", subcore_axis_name="s")
    gathered = pl.kernel(
        _sc_gather_body,
        mesh=mesh,
        out_type=jax.ShapeDtypeStruct((_NW * _CHUNKS_PER_W, _CHUNK, _D),
                                      jnp.float32),
        scratch_types=[
            pltpu.VMEM((_CHUNKS_PER_W, _CHUNK), jnp.int32),
            pltpu.VMEM((_CHUNKS_PER_W, _CHUNK, _D), jnp.float32),
            pltpu.SemaphoreType.DMA,
        ],
        compiler_params=pltpu.CompilerParams(use_tc_tiling_on_sc=False),
    )(codebook, q_pad)
    return gathered.reshape(_N_PAD, _D)[:_N]


def kernel(imgs, enc_w1, enc_b1, enc_w2, enc_b2, codebook, dec_w1, dec_b1,
           dec_w2, dec_b2):
    z_e = jax.nn.relu(_conv(imgs, enc_w1, enc_b1, 2))
    z_e = jax.nn.relu(_conv(z_e, enc_w2, enc_b2, 2))
    z = jnp.transpose(z_e, (0, 2, 3, 1))
    z_shape = z.shape
    flat = z.reshape(-1, _D)

    flat_sq = jnp.sum(flat ** 2, axis=1, keepdims=True)
    cb_sq = jnp.sum(codebook ** 2, axis=1).reshape(1, _K)

    q, loss_sum = _vq_argmin(flat, flat_sq, codebook, cb_sq)
    z_q_flat = _sc_gather(codebook, q)

    vq_loss = loss_sum * (2.0 / (_N * _D))

    z_q = z_q_flat.reshape(z_shape)
    encoded = jnp.transpose(z_q, (0, 3, 1, 2))
    d = jax.nn.relu(_conv_transpose(encoded, dec_w1, dec_b1))
    decoded = jax.nn.relu(_conv_transpose(d, dec_w2, dec_b2))
    return encoded, decoded, vq_loss
